# baseline (speedup 1.0000x reference)
import functools

import jax
import jax.numpy as jnp
from jax import lax
from jax.experimental import pallas as pl
from jax.experimental.pallas import tpu as pltpu
from jax.experimental.pallas import tpu_sc as plsc

N_NODES = 10000
D_FEAT = 128
AUGD = 144
ROWS = 10112
N_EDGES = 320000
NC, NS = 2, 16
NW = NC * NS
K = 64
BC = 8
CPT = 320
NF = 160
NS1 = CPT - NF
CH = NS * CPT
NEP = CH * K
STRIPE = ROWS // NS

IN_DIM = 2 * D_FEAT
H2 = 192
EMB = 128


@functools.cache
def _make_sc_agg():
    mesh = plsc.VectorSubcoreMesh(
        core_axis_name="c", subcore_axis_name="s",
        num_cores=NC, num_subcores=NS)

    @functools.partial(
        pl.kernel,
        out_type=jax.ShapeDtypeStruct((NC, ROWS, AUGD), jnp.float32),
        mesh=mesh,
        scratch_types=[
            pltpu.VMEM((4 * BC, K), jnp.int32),
            pltpu.VMEM((2, K, AUGD), jnp.float32),
            pltpu.VMEM_SHARED((ROWS, AUGD), jnp.float32),
            pltpu.SemaphoreType.DMA((2,)),
            pltpu.SemaphoreType.DMA((2,)),
        ],
        compiler_params=pltpu.CompilerParams(use_tc_tiling_on_sc=False),
    )
    def sc_agg(xaug_hbm, epk_hbm, parts_out, idxb, datab, acc, sg, si):
        c = lax.axis_index("c")
        s = lax.axis_index("s")
        nbf, nbs = NF // BC, NS1 // BC
        bstart = jnp.where(c == 0, s * nbf, NS * nbf + s * nbs)
        nb = jnp.where(c == 0, nbf, nbs)

        zb = datab.at[0]

        def _zrow(i, _):
            for g in range(AUGD // 16):
                zb[i, pl.ds(g * 16, 16)] = jnp.zeros((16,), jnp.float32)
            return _
        lax.fori_loop(0, K, _zrow, None)
        for kk in range(STRIPE // K):
            pltpu.sync_copy(zb, acc.at[pl.ds(s * STRIPE + kk * K, K)])
        rem = STRIPE % K
        if rem:
            pltpu.sync_copy(zb.at[pl.ds(0, rem)],
                            acc.at[pl.ds(s * STRIPE + (STRIPE // K) * K, rem)])
        plsc.subcore_barrier()

        pltpu.sync_copy(epk_hbm.at[bstart], idxb.at[pl.ds(0, 2 * BC)])
        pltpu.async_copy(xaug_hbm.at[idxb.at[0]], datab.at[0], sg.at[0])

        def body(b, _):
            slot = lax.rem(b, 2)
            nxt = 1 - slot
            base = slot * 2 * BC
            nbase = nxt * 2 * BC

            @pl.when(b + 1 < nb)
            def _():
                pltpu.async_copy(epk_hbm.at[bstart + b + 1],
                                 idxb.at[pl.ds(nbase, 2 * BC)], si.at[nxt])

            for t in range(BC):
                bt = t % 2
                if t < BC - 1:
                    pltpu.async_copy(xaug_hbm.at[idxb.at[base + 2 * (t + 1)]],
                                     datab.at[1 - bt], sg.at[1 - bt])
                else:
                    @pl.when(b + 1 < nb)
                    def _():
                        pltpu.make_async_copy(
                            epk_hbm.at[bstart], idxb.at[pl.ds(nbase, 2 * BC)],
                            si.at[nxt]).wait()
                        pltpu.async_copy(xaug_hbm.at[idxb.at[nbase]],
                                         datab.at[0], sg.at[0])
                pltpu.make_async_copy(xaug_hbm.at[pl.ds(0, K)], datab.at[bt],
                                      sg.at[bt]).wait()
                pltpu.sync_copy(datab.at[bt], acc.at[idxb.at[base + 2 * t + 1]],
                                add=True)
            return _

        lax.fori_loop(0, nb, body, None)

        plsc.subcore_barrier()
        pltpu.sync_copy(acc.at[pl.ds(s * STRIPE, STRIPE)],
                        parts_out.at[c, pl.ds(s * STRIPE, STRIPE)])

    return sc_agg


def _tc_dense_body(x_ref, parts_ref, w1_ref, b1_ref, w2_ref, b2_ref,
                   w3_ref, b3_ref, w4_ref, b4_ref, enc_ref, dec_ref):
    xs = x_ref[...]
    p = parts_ref[0] + parts_ref[1]
    cnt = p[:, D_FEAT:D_FEAT + 1]
    agg = p[:, :D_FEAT] / jnp.maximum(cnt, 1.0)
    col = lax.broadcasted_iota(jnp.int32, xs.shape, 1)
    xz = jnp.where(col == 0, 0.0, xs)
    aggz = jnp.where(col == 0, 0.0, agg)
    w1 = w1_ref[...]
    h = jnp.maximum(
        jnp.dot(xz, w1[:D_FEAT], preferred_element_type=jnp.float32)
        + jnp.dot(aggz, w1[D_FEAT:], preferred_element_type=jnp.float32)
        + b1_ref[...], 0.0)
    enc = jnp.dot(h, w2_ref[...], preferred_element_type=jnp.float32) + b2_ref[...]
    enc_ref[...] = enc
    h2 = jnp.maximum(
        jnp.dot(enc, w3_ref[...], preferred_element_type=jnp.float32)
        + b3_ref[...], 0.0)
    dec_ref[...] = (jnp.dot(h2, w4_ref[...], preferred_element_type=jnp.float32)
                    + b4_ref[...])


_TC_R = 1008


def _tc_dense(xp, parts, W_enc1, b_enc1, W_enc3, b_enc3,
              W_dec1, b_dec1, W_dec3, b_dec3):
    grid = (-(-N_NODES // _TC_R),)
    fixed = lambda i: (0, 0)
    enc, dec = pl.pallas_call(
        _tc_dense_body,
        grid=grid,
        in_specs=[
            pl.BlockSpec((_TC_R, D_FEAT), lambda i: (i, 0)),
            pl.BlockSpec((NC, _TC_R, AUGD), lambda i: (0, i, 0)),
            pl.BlockSpec((IN_DIM, H2), fixed),
            pl.BlockSpec((1, H2), fixed),
            pl.BlockSpec((H2, EMB), fixed),
            pl.BlockSpec((1, EMB), fixed),
            pl.BlockSpec((EMB, H2), fixed),
            pl.BlockSpec((1, H2), fixed),
            pl.BlockSpec((H2, IN_DIM), fixed),
            pl.BlockSpec((1, IN_DIM), fixed),
        ],
        out_specs=[
            pl.BlockSpec((_TC_R, EMB), lambda i: (i, 0)),
            pl.BlockSpec((_TC_R, IN_DIM), lambda i: (i, 0)),
        ],
        out_shape=[
            jax.ShapeDtypeStruct((N_NODES, EMB), jnp.float32),
            jax.ShapeDtypeStruct((N_NODES, IN_DIM), jnp.float32),
        ],
    )(xp, parts, W_enc1, b_enc1.reshape(1, H2), W_enc3, b_enc3.reshape(1, EMB),
      W_dec1, b_dec1.reshape(1, H2), W_dec3, b_dec3.reshape(1, IN_DIM))
    return enc, dec


def kernel(x, edge_index, W_enc1, b_enc1, W_enc3, b_enc3,
           W_dec1, b_dec1, W_dec3, b_dec3):
    xaug = jnp.concatenate(
        [x, jnp.ones((N_NODES, 1), jnp.float32),
         jnp.zeros((N_NODES, AUGD - D_FEAT - 1), jnp.float32)], axis=1)
    src = edge_index[0]
    dst = edge_index[1]
    pad = NEP - N_EDGES
    srcp = jnp.concatenate([src, jnp.zeros((pad,), jnp.int32)]).reshape(CH, 1, K)
    pad_dst = N_NODES + jnp.arange(pad, dtype=jnp.int32) % (ROWS - N_NODES)
    dstp = jnp.concatenate([dst, pad_dst]).reshape(CH, 1, K)
    epk = jnp.concatenate([srcp, dstp], axis=1).reshape(CH // BC, 2 * BC, K)

    parts = _make_sc_agg()(xaug, epk)

    enc, dec = _tc_dense(x, parts, W_enc1, b_enc1, W_enc3, b_enc3,
                         W_dec1, b_dec1, W_dec3, b_dec3)
    return enc, dec


# speedup vs baseline: 2.1293x; 2.1293x over previous
import functools

import jax
import jax.numpy as jnp
from jax import lax
from jax.experimental import pallas as pl
from jax.experimental.pallas import tpu as pltpu
from jax.experimental.pallas import tpu_sc as plsc

N_NODES = 10000
D_FEAT = 128
AUGD = 144
ROWS = 10112
N_EDGES = 320000
NC, NS = 2, 16
NW = NC * NS
K = 64
NCH0A = 158
NCH0B = 46
NCH1 = 110
CPT = NCH0A + NCH0B + NCH1
CH = NS * CPT
NEP = CH * K
STRIPE = ROWS // NS

IN_DIM = 2 * D_FEAT
H2 = 192
EMB = 128


@functools.cache
def _make_sc_agg():
    mesh = plsc.VectorSubcoreMesh(
        core_axis_name="c", subcore_axis_name="s",
        num_cores=NC, num_subcores=NS)

    @functools.partial(
        pl.kernel,
        out_type=jax.ShapeDtypeStruct((NC, ROWS, AUGD), jnp.float32),
        mesh=mesh,
        scratch_types=[
            pltpu.VMEM((NCH0A, K), jnp.int32),
            pltpu.VMEM((NCH0A, K), jnp.int32),
            pltpu.VMEM((K, AUGD), jnp.float32),
            pltpu.VMEM((K, AUGD), jnp.float32),
            pltpu.VMEM_SHARED((ROWS, AUGD), jnp.float32),
            pltpu.SemaphoreType.DMA,
            pltpu.SemaphoreType.DMA,
        ],
        compiler_params=pltpu.CompilerParams(use_tc_tiling_on_sc=False),
    )
    def sc_agg(xaug_hbm, src_hbm, dst_hbm, parts_out,
               sidx, didx, buf0, buf1, acc, sem0, sem1):
        c = lax.axis_index("c")
        s = lax.axis_index("s")

        def _zrow(i, _):
            for g in range(AUGD // 16):
                buf0[i, pl.ds(g * 16, 16)] = jnp.zeros((16,), jnp.float32)
            return _
        lax.fori_loop(0, K, _zrow, None)
        for kk in range(STRIPE // K):
            pltpu.sync_copy(buf0, acc.at[pl.ds(s * STRIPE + kk * K, K)])
        rem = STRIPE % K
        if rem:
            pltpu.sync_copy(buf0.at[pl.ds(0, rem)],
                            acc.at[pl.ds(s * STRIPE + (STRIPE // K) * K, rem)])
        plsc.subcore_barrier()

        def run_range(row0, nch):
            pltpu.sync_copy(src_hbm.at[pl.ds(row0, nch)],
                            sidx.at[pl.ds(0, nch)])
            pltpu.sync_copy(dst_hbm.at[pl.ds(row0, nch)],
                            didx.at[pl.ds(0, nch)])
            pltpu.async_copy(xaug_hbm.at[sidx.at[0]], buf0, sem0)

            def body(i, _):
                j = 2 * i
                pltpu.async_copy(xaug_hbm.at[sidx.at[j + 1]], buf1, sem1)
                pltpu.make_async_copy(xaug_hbm.at[sidx.at[j]], buf0,
                                      sem0).wait()
                pltpu.sync_copy(buf0, acc.at[didx.at[j]], add=True)

                @pl.when(j + 2 < nch)
                def _():
                    pltpu.async_copy(xaug_hbm.at[sidx.at[j + 2]], buf0, sem0)

                pltpu.make_async_copy(xaug_hbm.at[sidx.at[j + 1]], buf1,
                                      sem1).wait()
                pltpu.sync_copy(buf1, acc.at[didx.at[j + 1]], add=True)
                return _

            lax.fori_loop(0, nch // 2, body, None)

        @pl.when(c == 0)
        def _():
            run_range(s * NCH0A, NCH0A)
            run_range(NS * NCH0A + s * NCH0B, NCH0B)

        @pl.when(c == 1)
        def _():
            run_range(NS * (NCH0A + NCH0B) + s * NCH1, NCH1)

        plsc.subcore_barrier()
        pltpu.sync_copy(acc.at[pl.ds(s * STRIPE, STRIPE)],
                        parts_out.at[c, pl.ds(s * STRIPE, STRIPE)])

    return sc_agg


def _tc_dense_body(x_ref, parts_ref, w1_ref, b1_ref, w2_ref, b2_ref,
                   w3_ref, b3_ref, w4_ref, b4_ref, enc_ref, dec_ref):
    xs = x_ref[...]
    p = parts_ref[0] + parts_ref[1]
    cnt = p[:, D_FEAT:D_FEAT + 1]
    agg = p[:, :D_FEAT] / jnp.maximum(cnt, 1.0)
    col = lax.broadcasted_iota(jnp.int32, xs.shape, 1)
    xz = jnp.where(col == 0, 0.0, xs)
    aggz = jnp.where(col == 0, 0.0, agg)
    w1 = w1_ref[...]
    h = jnp.maximum(
        jnp.dot(xz, w1[:D_FEAT], preferred_element_type=jnp.float32)
        + jnp.dot(aggz, w1[D_FEAT:], preferred_element_type=jnp.float32)
        + b1_ref[...], 0.0)
    enc = jnp.dot(h, w2_ref[...], preferred_element_type=jnp.float32) + b2_ref[...]
    enc_ref[...] = enc
    h2 = jnp.maximum(
        jnp.dot(enc, w3_ref[...], preferred_element_type=jnp.float32)
        + b3_ref[...], 0.0)
    dec_ref[...] = (jnp.dot(h2, w4_ref[...], preferred_element_type=jnp.float32)
                    + b4_ref[...])


_TC_R = 1008


def _tc_dense(xp, parts, W_enc1, b_enc1, W_enc3, b_enc3,
              W_dec1, b_dec1, W_dec3, b_dec3):
    grid = (-(-N_NODES // _TC_R),)
    fixed = lambda i: (0, 0)
    enc, dec = pl.pallas_call(
        _tc_dense_body,
        grid=grid,
        in_specs=[
            pl.BlockSpec((_TC_R, D_FEAT), lambda i: (i, 0)),
            pl.BlockSpec((NC, _TC_R, AUGD), lambda i: (0, i, 0)),
            pl.BlockSpec((IN_DIM, H2), fixed),
            pl.BlockSpec((1, H2), fixed),
            pl.BlockSpec((H2, EMB), fixed),
            pl.BlockSpec((1, EMB), fixed),
            pl.BlockSpec((EMB, H2), fixed),
            pl.BlockSpec((1, H2), fixed),
            pl.BlockSpec((H2, IN_DIM), fixed),
            pl.BlockSpec((1, IN_DIM), fixed),
        ],
        out_specs=[
            pl.BlockSpec((_TC_R, EMB), lambda i: (i, 0)),
            pl.BlockSpec((_TC_R, IN_DIM), lambda i: (i, 0)),
        ],
        out_shape=[
            jax.ShapeDtypeStruct((N_NODES, EMB), jnp.float32),
            jax.ShapeDtypeStruct((N_NODES, IN_DIM), jnp.float32),
        ],
    )(xp, parts, W_enc1, b_enc1.reshape(1, H2), W_enc3, b_enc3.reshape(1, EMB),
      W_dec1, b_dec1.reshape(1, H2), W_dec3, b_dec3.reshape(1, IN_DIM))
    return enc, dec


def kernel(x, edge_index, W_enc1, b_enc1, W_enc3, b_enc3,
           W_dec1, b_dec1, W_dec3, b_dec3):
    xaug = jnp.concatenate(
        [x, jnp.ones((N_NODES, 1), jnp.float32),
         jnp.zeros((N_NODES, AUGD - D_FEAT - 1), jnp.float32)], axis=1)
    src = edge_index[0]
    dst = edge_index[1]
    pad = NEP - N_EDGES
    srcp = jnp.concatenate([src, jnp.zeros((pad,), jnp.int32)]).reshape(CH, K)
    pad_dst = N_NODES + jnp.arange(pad, dtype=jnp.int32) % (ROWS - N_NODES)
    dstp = jnp.concatenate([dst, pad_dst]).reshape(CH, K)

    parts = _make_sc_agg()(xaug, srcp, dstp)

    enc, dec = _tc_dense(x, parts, W_enc1, b_enc1, W_enc3, b_enc3,
                         W_dec1, b_dec1, W_dec3, b_dec3)
    return enc, dec


# speedup vs baseline: 2.6447x; 1.2420x over previous
import functools

import jax
import jax.numpy as jnp
from jax import lax
from jax.experimental import pallas as pl
from jax.experimental.pallas import tpu as pltpu
from jax.experimental.pallas import tpu_sc as plsc

N_NODES = 10000
D_FEAT = 128
ROWS = 10112
N_EDGES = 320000
NC, NS = 2, 16
NW = NC * NS
K = 64
NCH0A = 158
NCH0B = 34
NCH1 = 122
CPT = NCH0A + NCH0B + NCH1
CH = NS * CPT
NEP = CH * K
STRIPE = ROWS // NS

IN_DIM = 2 * D_FEAT
H2 = 192
EMB = 128


@functools.cache
def _make_sc_agg():
    mesh = plsc.VectorSubcoreMesh(
        core_axis_name="c", subcore_axis_name="s",
        num_cores=NC, num_subcores=NS)

    @functools.partial(
        pl.kernel,
        out_type=(jax.ShapeDtypeStruct((NC, ROWS, D_FEAT), jnp.float32),
                  jax.ShapeDtypeStruct((NW, ROWS), jnp.float32)),
        mesh=mesh,
        scratch_types=[
            pltpu.VMEM((NCH0A, K), jnp.int32),
            pltpu.VMEM((NCH0A, K), jnp.int32),
            pltpu.VMEM((K, D_FEAT), jnp.float32),
            pltpu.VMEM((K, D_FEAT), jnp.float32),
            pltpu.VMEM((ROWS,), jnp.float32),
            pltpu.VMEM_SHARED((ROWS, D_FEAT), jnp.float32),
            pltpu.SemaphoreType.DMA,
            pltpu.SemaphoreType.DMA,
        ],
        compiler_params=pltpu.CompilerParams(use_tc_tiling_on_sc=False,
                                             needs_layout_passes=False),
    )
    def sc_agg(x_hbm, src_hbm, dst_hbm, parts_out, cnt_out,
               sidx, didx, buf0, buf1, cnts, acc, sem0, sem1):
        c = lax.axis_index("c")
        s = lax.axis_index("s")
        ones = jnp.ones((16,), jnp.float32)
        zeros = jnp.zeros((16,), jnp.float32)

        def _zrow(i, _):
            for g in range(D_FEAT // 16):
                buf0[i, pl.ds(g * 16, 16)] = zeros
            return _
        lax.fori_loop(0, K, _zrow, None)

        def _zcnt(i, _):
            cnts[pl.ds(i * 16, 16)] = zeros
            return _
        lax.fori_loop(0, ROWS // 16, _zcnt, None)

        for kk in range(STRIPE // K):
            pltpu.sync_copy(buf0, acc.at[pl.ds(s * STRIPE + kk * K, K)])
        rem = STRIPE % K
        if rem:
            pltpu.sync_copy(buf0.at[pl.ds(0, rem)],
                            acc.at[pl.ds(s * STRIPE + (STRIPE // K) * K, rem)])
        plsc.subcore_barrier()

        def count_chunk(j):
            for g in range(K // 16):
                idx16 = didx[j, pl.ds(g * 16, 16)]
                plsc.addupdate_scatter(cnts, [idx16], ones)

        def run_range(row0, nch):
            pltpu.sync_copy(src_hbm.at[pl.ds(row0, nch)],
                            sidx.at[pl.ds(0, nch)])
            pltpu.sync_copy(dst_hbm.at[pl.ds(row0, nch)],
                            didx.at[pl.ds(0, nch)])
            pltpu.async_copy(x_hbm.at[sidx.at[0]], buf0, sem0)

            def body(i, _):
                j = 2 * i
                pltpu.async_copy(x_hbm.at[sidx.at[j + 1]], buf1, sem1)
                count_chunk(j)
                pltpu.make_async_copy(x_hbm.at[sidx.at[j]], buf0,
                                      sem0).wait()
                pltpu.sync_copy(buf0, acc.at[didx.at[j]], add=True)

                @pl.when(j + 2 < nch)
                def _():
                    pltpu.async_copy(x_hbm.at[sidx.at[j + 2]], buf0, sem0)

                count_chunk(j + 1)
                pltpu.make_async_copy(x_hbm.at[sidx.at[j + 1]], buf1,
                                      sem1).wait()
                pltpu.sync_copy(buf1, acc.at[didx.at[j + 1]], add=True)
                return _

            lax.fori_loop(0, nch // 2, body, None)

        @pl.when(c == 0)
        def _():
            run_range(s * NCH0A, NCH0A)
            run_range(NS * NCH0A + s * NCH0B, NCH0B)

        @pl.when(c == 1)
        def _():
            run_range(NS * (NCH0A + NCH0B) + s * NCH1, NCH1)

        plsc.subcore_barrier()
        pltpu.sync_copy(acc.at[pl.ds(s * STRIPE, STRIPE)],
                        parts_out.at[c, pl.ds(s * STRIPE, STRIPE)])
        pltpu.sync_copy(cnts, cnt_out.at[c * NS + s])

    return sc_agg


def _tc_dense_body(x_ref, parts_ref, cnt_ref, w1_ref, b1_ref, w2_ref, b2_ref,
                   w3_ref, b3_ref, w4_ref, b4_ref, enc_ref, dec_ref):
    xs = x_ref[...]
    p = parts_ref[0] + parts_ref[1]
    cnt = jnp.sum(cnt_ref[...], axis=1, keepdims=True)
    agg = p / jnp.maximum(cnt, 1.0)
    col = lax.broadcasted_iota(jnp.int32, xs.shape, 1)
    xz = jnp.where(col == 0, 0.0, xs)
    aggz = jnp.where(col == 0, 0.0, agg)
    w1 = w1_ref[...]
    h = jnp.maximum(
        jnp.dot(xz, w1[:D_FEAT], preferred_element_type=jnp.float32)
        + jnp.dot(aggz, w1[D_FEAT:], preferred_element_type=jnp.float32)
        + b1_ref[...], 0.0)
    enc = jnp.dot(h, w2_ref[...], preferred_element_type=jnp.float32) + b2_ref[...]
    enc_ref[...] = enc
    h2 = jnp.maximum(
        jnp.dot(enc, w3_ref[...], preferred_element_type=jnp.float32)
        + b3_ref[...], 0.0)
    dec_ref[...] = (jnp.dot(h2, w4_ref[...], preferred_element_type=jnp.float32)
                    + b4_ref[...])


_TC_R = 1008


def _tc_dense(xp, parts, cntT, W_enc1, b_enc1, W_enc3, b_enc3,
              W_dec1, b_dec1, W_dec3, b_dec3):
    grid = (-(-N_NODES // _TC_R),)
    fixed = lambda i: (0, 0)
    enc, dec = pl.pallas_call(
        _tc_dense_body,
        grid=grid,
        in_specs=[
            pl.BlockSpec((_TC_R, D_FEAT), lambda i: (i, 0)),
            pl.BlockSpec((NC, _TC_R, D_FEAT), lambda i: (0, i, 0)),
            pl.BlockSpec((_TC_R, NW), lambda i: (i, 0)),
            pl.BlockSpec((IN_DIM, H2), fixed),
            pl.BlockSpec((1, H2), fixed),
            pl.BlockSpec((H2, EMB), fixed),
            pl.BlockSpec((1, EMB), fixed),
            pl.BlockSpec((EMB, H2), fixed),
            pl.BlockSpec((1, H2), fixed),
            pl.BlockSpec((H2, IN_DIM), fixed),
            pl.BlockSpec((1, IN_DIM), fixed),
        ],
        out_specs=[
            pl.BlockSpec((_TC_R, EMB), lambda i: (i, 0)),
            pl.BlockSpec((_TC_R, IN_DIM), lambda i: (i, 0)),
        ],
        out_shape=[
            jax.ShapeDtypeStruct((N_NODES, EMB), jnp.float32),
            jax.ShapeDtypeStruct((N_NODES, IN_DIM), jnp.float32),
        ],
    )(xp, parts, cntT, W_enc1, b_enc1.reshape(1, H2), W_enc3,
      b_enc3.reshape(1, EMB), W_dec1, b_dec1.reshape(1, H2), W_dec3,
      b_dec3.reshape(1, IN_DIM))
    return enc, dec


def kernel(x, edge_index, W_enc1, b_enc1, W_enc3, b_enc3,
           W_dec1, b_dec1, W_dec3, b_dec3):
    src = edge_index[0]
    dst = edge_index[1]
    pad = NEP - N_EDGES
    srcp = jnp.concatenate([src, jnp.zeros((pad,), jnp.int32)]).reshape(CH, K)
    pad_dst = N_NODES + jnp.arange(pad, dtype=jnp.int32) % (ROWS - N_NODES)
    dstp = jnp.concatenate([dst, pad_dst]).reshape(CH, K)

    parts, cnt = _make_sc_agg()(x, srcp, dstp)

    enc, dec = _tc_dense(x, parts, cnt.T, W_enc1, b_enc1, W_enc3, b_enc3,
                         W_dec1, b_dec1, W_dec3, b_dec3)
    return enc, dec


# speedup vs baseline: 3.0045x; 1.1360x over previous
import functools

import jax
import jax.numpy as jnp
from jax import lax
from jax.experimental import pallas as pl
from jax.experimental.pallas import tpu as pltpu
from jax.experimental.pallas import tpu_sc as plsc

N_NODES = 10000
D_FEAT = 128
ROWS = 10112
N_EDGES = 320000
NC, NS = 2, 16
NW = NC * NS
K = 80
CH = N_EDGES // K
NCHA = 76
NCH0 = (NCHA, NCHA)
NCH1 = (NCHA, 22)
STRIPE = ROWS // NS

IN_DIM = 2 * D_FEAT
H2 = 192
EMB = 128


@functools.cache
def _make_sc_agg():
    mesh = plsc.VectorSubcoreMesh(
        core_axis_name="c", subcore_axis_name="s",
        num_cores=NC, num_subcores=NS)

    @functools.partial(
        pl.kernel,
        out_type=(jax.ShapeDtypeStruct((NC, ROWS, D_FEAT), jnp.float32),
                  jax.ShapeDtypeStruct((NW, ROWS), jnp.float32)),
        mesh=mesh,
        scratch_types=[
            pltpu.VMEM((NCHA, K), jnp.int32),
            pltpu.VMEM((NCHA, K), jnp.int32),
            pltpu.VMEM((K, D_FEAT), jnp.float32),
            pltpu.VMEM((K, D_FEAT), jnp.float32),
            pltpu.VMEM((ROWS,), jnp.float32),
            pltpu.VMEM_SHARED((ROWS, D_FEAT), jnp.float32),
            pltpu.SemaphoreType.DMA,
            pltpu.SemaphoreType.DMA,
        ],
        compiler_params=pltpu.CompilerParams(use_tc_tiling_on_sc=False,
                                             needs_layout_passes=False),
    )
    def sc_agg(x_hbm, src_hbm, dst_hbm, parts_out, cnt_out,
               sidx, didx, buf0, buf1, cnts, acc, sem0, sem1):
        c = lax.axis_index("c")
        s = lax.axis_index("s")
        ones = jnp.ones((16,), jnp.float32)
        zeros = jnp.zeros((16,), jnp.float32)

        def _zrow(i, _):
            for g in range(D_FEAT // 16):
                buf0[i, pl.ds(g * 16, 16)] = zeros
            return _
        lax.fori_loop(0, K, _zrow, None)

        def _zcnt(i, _):
            cnts[pl.ds(i * 16, 16)] = zeros
            return _
        lax.fori_loop(0, ROWS // 16, _zcnt, None)

        for kk in range(STRIPE // K):
            pltpu.sync_copy(buf0, acc.at[pl.ds(s * STRIPE + kk * K, K)])
        rem = STRIPE % K
        if rem:
            pltpu.sync_copy(buf0.at[pl.ds(0, rem)],
                            acc.at[pl.ds(s * STRIPE + (STRIPE // K) * K, rem)])
        plsc.subcore_barrier()

        def count_chunk(j):
            for g in range(K // 16):
                idx16 = didx[j, pl.ds(g * 16, 16)]
                plsc.addupdate_scatter(cnts, [idx16], ones)

        def run_range(row0, nch):
            pltpu.sync_copy(src_hbm.at[pl.ds(row0, nch)],
                            sidx.at[pl.ds(0, nch)])
            pltpu.sync_copy(dst_hbm.at[pl.ds(row0, nch)],
                            didx.at[pl.ds(0, nch)])
            pltpu.async_copy(x_hbm.at[sidx.at[0]], buf0, sem0)

            def body(i, _):
                j = 2 * i
                pltpu.async_copy(x_hbm.at[sidx.at[j + 1]], buf1, sem1)
                count_chunk(j)
                pltpu.make_async_copy(x_hbm.at[sidx.at[j]], buf0,
                                      sem0).wait()
                pltpu.sync_copy(buf0, acc.at[didx.at[j]], add=True)

                @pl.when(j + 2 < nch)
                def _():
                    pltpu.async_copy(x_hbm.at[sidx.at[j + 2]], buf0, sem0)

                count_chunk(j + 1)
                pltpu.make_async_copy(x_hbm.at[sidx.at[j + 1]], buf1,
                                      sem1).wait()
                pltpu.sync_copy(buf1, acc.at[didx.at[j + 1]], add=True)
                return _

            lax.fori_loop(0, nch // 2, body, None)

        n0 = sum(NCH0)
        @pl.when(c == 0)
        def _():
            run_range(s * NCH0[0], NCH0[0])
            run_range(NS * NCH0[0] + s * NCH0[1], NCH0[1])

        @pl.when(c == 1)
        def _():
            run_range(NS * n0 + s * NCH1[0], NCH1[0])
            run_range(NS * (n0 + NCH1[0]) + s * NCH1[1], NCH1[1])

        plsc.subcore_barrier()
        pltpu.sync_copy(acc.at[pl.ds(s * STRIPE, STRIPE)],
                        parts_out.at[c, pl.ds(s * STRIPE, STRIPE)])
        pltpu.sync_copy(cnts, cnt_out.at[c * NS + s])

    return sc_agg


def _tc_dense_body(x_ref, parts_ref, cnt_ref, w1_ref, b1_ref, w2_ref, b2_ref,
                   w3_ref, b3_ref, w4_ref, b4_ref, enc_ref, dec_ref):
    xs = x_ref[...]
    p = parts_ref[0] + parts_ref[1]
    cnt = jnp.sum(cnt_ref[...], axis=1, keepdims=True)
    agg = p / jnp.maximum(cnt, 1.0)
    col = lax.broadcasted_iota(jnp.int32, xs.shape, 1)
    xz = jnp.where(col == 0, 0.0, xs)
    aggz = jnp.where(col == 0, 0.0, agg)
    w1 = w1_ref[...]
    h = jnp.maximum(
        jnp.dot(xz, w1[:D_FEAT], preferred_element_type=jnp.float32)
        + jnp.dot(aggz, w1[D_FEAT:], preferred_element_type=jnp.float32)
        + b1_ref[...], 0.0)
    enc = jnp.dot(h, w2_ref[...], preferred_element_type=jnp.float32) + b2_ref[...]
    enc_ref[...] = enc
    h2 = jnp.maximum(
        jnp.dot(enc, w3_ref[...], preferred_element_type=jnp.float32)
        + b3_ref[...], 0.0)
    dec_ref[...] = (jnp.dot(h2, w4_ref[...], preferred_element_type=jnp.float32)
                    + b4_ref[...])


_TC_R = 1008


def _tc_dense(xp, parts, cntT, W_enc1, b_enc1, W_enc3, b_enc3,
              W_dec1, b_dec1, W_dec3, b_dec3):
    grid = (-(-N_NODES // _TC_R),)
    fixed = lambda i: (0, 0)
    enc, dec = pl.pallas_call(
        _tc_dense_body,
        grid=grid,
        in_specs=[
            pl.BlockSpec((_TC_R, D_FEAT), lambda i: (i, 0)),
            pl.BlockSpec((NC, _TC_R, D_FEAT), lambda i: (0, i, 0)),
            pl.BlockSpec((_TC_R, NW), lambda i: (i, 0)),
            pl.BlockSpec((IN_DIM, H2), fixed),
            pl.BlockSpec((1, H2), fixed),
            pl.BlockSpec((H2, EMB), fixed),
            pl.BlockSpec((1, EMB), fixed),
            pl.BlockSpec((EMB, H2), fixed),
            pl.BlockSpec((1, H2), fixed),
            pl.BlockSpec((H2, IN_DIM), fixed),
            pl.BlockSpec((1, IN_DIM), fixed),
        ],
        out_specs=[
            pl.BlockSpec((_TC_R, EMB), lambda i: (i, 0)),
            pl.BlockSpec((_TC_R, IN_DIM), lambda i: (i, 0)),
        ],
        out_shape=[
            jax.ShapeDtypeStruct((N_NODES, EMB), jnp.float32),
            jax.ShapeDtypeStruct((N_NODES, IN_DIM), jnp.float32),
        ],
    )(xp, parts, cntT, W_enc1, b_enc1.reshape(1, H2), W_enc3,
      b_enc3.reshape(1, EMB), W_dec1, b_dec1.reshape(1, H2), W_dec3,
      b_dec3.reshape(1, IN_DIM))
    return enc, dec


def kernel(x, edge_index, W_enc1, b_enc1, W_enc3, b_enc3,
           W_dec1, b_dec1, W_dec3, b_dec3):
    srcp = edge_index[0].reshape(CH, K)
    dstp = edge_index[1].reshape(CH, K)

    parts, cnt = _make_sc_agg()(x, srcp, dstp)

    enc, dec = _tc_dense(x, parts, cnt.T, W_enc1, b_enc1, W_enc3, b_enc3,
                         W_dec1, b_dec1, W_dec3, b_dec3)
    return enc, dec


# speedup vs baseline: 3.3445x; 1.1132x over previous
import functools

import jax
import jax.numpy as jnp
from jax import lax
from jax.experimental import pallas as pl
from jax.experimental.pallas import tpu as pltpu
from jax.experimental.pallas import tpu_sc as plsc

N_NODES = 10000
D_FEAT = 128
ROWS = 10112
N_EDGES = 320000
NC, NS = 2, 16
NW = NC * NS
K = 80
CH = N_EDGES // K
NCHA = 76
NCH0 = (NCHA, 54)
NCH1 = (NCHA, 44)
STRIPE = ROWS // NS
CROWS = 80

IN_DIM = 2 * D_FEAT
H2 = 192
EMB = 128


@functools.cache
def _make_sc_agg():
    mesh = plsc.VectorSubcoreMesh(
        core_axis_name="c", subcore_axis_name="s",
        num_cores=NC, num_subcores=NS)

    @functools.partial(
        pl.kernel,
        out_type=(jax.ShapeDtypeStruct((NC, ROWS, D_FEAT), jnp.float32),
                  jax.ShapeDtypeStruct((NC, CROWS, 128), jnp.float32)),
        mesh=mesh,
        scratch_types=[
            pltpu.VMEM((NCHA, K), jnp.int32),
            pltpu.VMEM((NCHA, K), jnp.int32),
            pltpu.VMEM((K, D_FEAT), jnp.float32),
            pltpu.VMEM((K, D_FEAT), jnp.float32),
            pltpu.VMEM((CROWS, 128), jnp.float32),
            pltpu.VMEM((CROWS,), jnp.int32),
            pltpu.VMEM_SHARED((ROWS, D_FEAT), jnp.float32),
            pltpu.VMEM_SHARED((CROWS, 128), jnp.float32),
            pltpu.SemaphoreType.DMA,
            pltpu.SemaphoreType.DMA,
        ],
        compiler_params=pltpu.CompilerParams(use_tc_tiling_on_sc=False,
                                             needs_layout_passes=False),
    )
    def sc_agg(x_hbm, src_hbm, dst_hbm, parts_out, cnt_out,
               sidx, didx, buf0, buf1, cnts, rowix, acc, csp, sem0, sem1):
        c = lax.axis_index("c")
        s = lax.axis_index("s")
        ones = jnp.ones((16,), jnp.float32)
        zeros = jnp.zeros((16,), jnp.float32)

        def _zrow(i, _):
            for g in range(D_FEAT // 16):
                buf0[i, pl.ds(g * 16, 16)] = zeros
            return _
        lax.fori_loop(0, K, _zrow, None)

        def _zcnt(i, _):
            for g in range(128 // 16):
                cnts[i, pl.ds(g * 16, 16)] = zeros
            return _
        lax.fori_loop(0, CROWS, _zcnt, None)
        iota16 = lax.broadcasted_iota(jnp.int32, (16,), 0)
        for g in range(CROWS // 16):
            rowix[pl.ds(g * 16, 16)] = iota16 + (g * 16)

        for kk in range(STRIPE // K):
            pltpu.sync_copy(buf0, acc.at[pl.ds(s * STRIPE + kk * K, K)])
        rem = STRIPE % K
        if rem:
            pltpu.sync_copy(buf0.at[pl.ds(0, rem)],
                            acc.at[pl.ds(s * STRIPE + (STRIPE // K) * K, rem)])

        @pl.when(s == 0)
        def _():
            pltpu.sync_copy(buf0, csp)
        plsc.subcore_barrier()

        def count_chunk(j):
            for g in range(K // 16):
                idx16 = didx[j, pl.ds(g * 16, 16)]
                plsc.addupdate_scatter(
                    cnts,
                    [lax.shift_right_logical(idx16, 7),
                     lax.bitwise_and(idx16, 127)], ones)

        def run_range(row0, nch):
            pltpu.sync_copy(src_hbm.at[pl.ds(row0, nch)],
                            sidx.at[pl.ds(0, nch)])
            pltpu.sync_copy(dst_hbm.at[pl.ds(row0, nch)],
                            didx.at[pl.ds(0, nch)])
            pltpu.async_copy(x_hbm.at[sidx.at[0]], buf0, sem0)

            def body(i, _):
                j = 2 * i
                pltpu.async_copy(x_hbm.at[sidx.at[j + 1]], buf1, sem1)
                count_chunk(j)
                pltpu.make_async_copy(x_hbm.at[sidx.at[j]], buf0,
                                      sem0).wait()
                pltpu.sync_copy(buf0, acc.at[didx.at[j]], add=True)

                @pl.when(j + 2 < nch)
                def _():
                    pltpu.async_copy(x_hbm.at[sidx.at[j + 2]], buf0, sem0)

                count_chunk(j + 1)
                pltpu.make_async_copy(x_hbm.at[sidx.at[j + 1]], buf1,
                                      sem1).wait()
                pltpu.sync_copy(buf1, acc.at[didx.at[j + 1]], add=True)
                return _

            lax.fori_loop(0, nch // 2, body, None)

        n0 = sum(NCH0)
        @pl.when(c == 0)
        def _():
            run_range(s * NCH0[0], NCH0[0])
            run_range(NS * NCH0[0] + s * NCH0[1], NCH0[1])

        @pl.when(c == 1)
        def _():
            run_range(NS * n0 + s * NCH1[0], NCH1[0])
            run_range(NS * (n0 + NCH1[0]) + s * NCH1[1], NCH1[1])

        plsc.subcore_barrier()
        pltpu.sync_copy(cnts, csp.at[rowix], add=True)
        pltpu.sync_copy(acc.at[pl.ds(s * STRIPE, STRIPE)],
                        parts_out.at[c, pl.ds(s * STRIPE, STRIPE)])
        plsc.subcore_barrier()

        @pl.when(s == 0)
        def _():
            pltpu.sync_copy(csp, cnt_out.at[c])

    return sc_agg


def _tc_dense_body(x_ref, parts_ref, cnt_ref, w1_ref, b1_ref, w2_ref, b2_ref,
                   w3_ref, b3_ref, w4_ref, b4_ref, enc_ref, dec_ref):
    xs = x_ref[...]
    p = parts_ref[0] + parts_ref[1]
    cnt = jnp.sum(cnt_ref[...], axis=1, keepdims=True)
    agg = p / jnp.maximum(cnt, 1.0)
    col = lax.broadcasted_iota(jnp.int32, xs.shape, 1)
    xz = jnp.where(col == 0, 0.0, xs)
    aggz = jnp.where(col == 0, 0.0, agg)
    w1 = w1_ref[...]
    h = jnp.maximum(
        jnp.dot(xz, w1[:D_FEAT], preferred_element_type=jnp.float32)
        + jnp.dot(aggz, w1[D_FEAT:], preferred_element_type=jnp.float32)
        + b1_ref[...], 0.0)
    enc = jnp.dot(h, w2_ref[...], preferred_element_type=jnp.float32) + b2_ref[...]
    enc_ref[...] = enc
    h2 = jnp.maximum(
        jnp.dot(enc, w3_ref[...], preferred_element_type=jnp.float32)
        + b3_ref[...], 0.0)
    dec_ref[...] = (jnp.dot(h2, w4_ref[...], preferred_element_type=jnp.float32)
                    + b4_ref[...])


_TC_R = 1008


def _tc_dense(xp, parts, cntT, W_enc1, b_enc1, W_enc3, b_enc3,
              W_dec1, b_dec1, W_dec3, b_dec3):
    grid = (-(-N_NODES // _TC_R),)
    fixed = lambda i: (0, 0)
    enc, dec = pl.pallas_call(
        _tc_dense_body,
        grid=grid,
        in_specs=[
            pl.BlockSpec((_TC_R, D_FEAT), lambda i: (i, 0)),
            pl.BlockSpec((NC, _TC_R, D_FEAT), lambda i: (0, i, 0)),
            pl.BlockSpec((_TC_R, NC), lambda i: (i, 0)),
            pl.BlockSpec((IN_DIM, H2), fixed),
            pl.BlockSpec((1, H2), fixed),
            pl.BlockSpec((H2, EMB), fixed),
            pl.BlockSpec((1, EMB), fixed),
            pl.BlockSpec((EMB, H2), fixed),
            pl.BlockSpec((1, H2), fixed),
            pl.BlockSpec((H2, IN_DIM), fixed),
            pl.BlockSpec((1, IN_DIM), fixed),
        ],
        out_specs=[
            pl.BlockSpec((_TC_R, EMB), lambda i: (i, 0)),
            pl.BlockSpec((_TC_R, IN_DIM), lambda i: (i, 0)),
        ],
        out_shape=[
            jax.ShapeDtypeStruct((N_NODES, EMB), jnp.float32),
            jax.ShapeDtypeStruct((N_NODES, IN_DIM), jnp.float32),
        ],
    )(xp, parts, cntT, W_enc1, b_enc1.reshape(1, H2), W_enc3,
      b_enc3.reshape(1, EMB), W_dec1, b_dec1.reshape(1, H2), W_dec3,
      b_dec3.reshape(1, IN_DIM))
    return enc, dec


def kernel(x, edge_index, W_enc1, b_enc1, W_enc3, b_enc3,
           W_dec1, b_dec1, W_dec3, b_dec3):
    srcp = edge_index[0].reshape(CH, K)
    dstp = edge_index[1].reshape(CH, K)

    parts, cnt = _make_sc_agg()(x, srcp, dstp)

    cntT = cnt.reshape(NC, CROWS * 128).T
    enc, dec = _tc_dense(x, parts, cntT, W_enc1, b_enc1, W_enc3, b_enc3,
                         W_dec1, b_dec1, W_dec3, b_dec3)
    return enc, dec
